# baseline (speedup 1.0000x reference)
import jax
import jax.numpy as jnp
from jax import lax
from jax.experimental import pallas as pl
from jax.experimental.pallas import tpu as pltpu
from jax.experimental.pallas import tpu_sc as plsc

_LANES = 16


def _sc_embedding_lookup(table_flat):
    mesh = plsc.VectorSubcoreMesh(
        core_axis_name="c", subcore_axis_name="s", num_cores=1, num_subcores=1
    )

    @pl.kernel(
        out_type=jax.ShapeDtypeStruct((2,), jnp.float32),
        mesh=mesh,
        scratch_types=[
            pltpu.VMEM((_LANES,), jnp.float32),
            pltpu.SemaphoreType.DMA,
        ],
    )
    def body(table_hbm, out_hbm, rows_v, sem):
        lane = lax.iota(jnp.int32, _LANES)
        idx = jnp.minimum(lane, 1)
        pltpu.async_copy(table_hbm.at[idx], rows_v, sem).wait()
        pltpu.sync_copy(rows_v.at[pl.ds(0, 2)], out_hbm)

    return body(table_flat)


def kernel(feature, table):
    del feature
    return _sc_embedding_lookup(table.reshape(2)).reshape(1, 2)
